# Initial kernel scaffold; baseline (speedup 1.0000x reference)
import jax
import jax.numpy as jnp
from jax.experimental import pallas as pl


def kernel(x, edge_index, Wl1, bl1, Wr1, Wl2, bl2, Wr2, Wl3, bl3, Wr3):
    raise NotImplementedError("write your pallas kernel here")


# speedup vs baseline: 5.0560x; 5.0560x over previous
import functools

import jax
import jax.numpy as jnp
from jax import lax
from jax.experimental import pallas as pl
from jax.experimental.pallas import tpu as pltpu
from jax.experimental.pallas import tpu_sc as plsc

NC = 2
NS = 16
NW = NC * NS
LANES = 16
CHUNK = 128
ZROWS = 16


def _fill_rows(ref, nrows, width, value):
    vec = jnp.full((LANES,), value, jnp.float32)

    def body(i, _):
        r = i // (width // LANES)
        c = i % (width // LANES)
        ref[r, pl.ds(c * LANES, LANES)] = vec
        return 0

    lax.fori_loop(0, nrows * (width // LANES), body, 0)


def _sc_mesh():
    return plsc.VectorSubcoreMesh(
        core_axis_name="c", subcore_axis_name="s",
        num_cores=NC, num_subcores=NS)


def _edge_loop(wid, n_chunks, body):
    n_full = n_chunks // NW
    n_extra = n_chunks % NW

    def step(k, _):
        c = wid + k * NW
        base = pl.multiple_of(c * CHUNK, CHUNK)
        body(base)
        return 0

    n_mine = n_full + jnp.where(wid < n_extra, 1, 0)
    lax.fori_loop(0, n_mine, step, 0)


def _zero_slice(zero_v, acc_sh, off, rows):
    def body(b, _):
        pltpu.sync_copy(zero_v, acc_sh.at[pl.ds(off + b * ZROWS, ZROWS)])
        return 0

    lax.fori_loop(0, rows // ZROWS, body, 0)


def _sc_aggregate(h, src, dst):
    n_nodes, d = h.shape
    e = src.shape[0]
    n_chunks = e // CHUNK
    rows_per_tile = n_nodes // NS

    @functools.partial(
        pl.kernel,
        out_type=jax.ShapeDtypeStruct((NC, n_nodes, d), jnp.float32),
        mesh=_sc_mesh(),
        scratch_types=[
            pltpu.VMEM((CHUNK,), jnp.int32),
            pltpu.VMEM((CHUNK,), jnp.int32),
            pltpu.VMEM((CHUNK, d), jnp.float32),
            pltpu.VMEM((ZROWS, d), jnp.float32),
            pltpu.VMEM_SHARED((n_nodes, d), jnp.float32),
            pltpu.SemaphoreType.DMA,
        ],
    )
    def k(h_hbm, src_hbm, dst_hbm, out_hbm,
          src_v, dst_v, rows_v, zero_v, acc_sh, sem):
        cid = lax.axis_index("c")
        sid = lax.axis_index("s")
        wid = sid * NC + cid
        _fill_rows(zero_v, ZROWS, d, 0.0)
        off = sid * rows_per_tile
        _zero_slice(zero_v, acc_sh, off, rows_per_tile)
        plsc.subcore_barrier()

        def body(base):
            pltpu.sync_copy(src_hbm.at[pl.ds(base, CHUNK)], src_v)
            pltpu.sync_copy(dst_hbm.at[pl.ds(base, CHUNK)], dst_v)
            pltpu.async_copy(h_hbm.at[src_v], rows_v, sem).wait()
            pltpu.sync_copy(rows_v, acc_sh.at[dst_v], add=True)

        _edge_loop(wid, n_chunks, body)
        plsc.subcore_barrier()
        pltpu.sync_copy(acc_sh.at[pl.ds(off, rows_per_tile)],
                        out_hbm.at[cid, pl.ds(off, rows_per_tile)])

    return k(h, src, dst)


def _tc_combine(aggp, degp, h, wl_t, bl, wr_t, last):
    n_nodes, d = h.shape
    blk = 640

    def body(agg_ref, deg_ref, h_ref, wl_ref, bl_ref, wr_ref, out_ref):
        agg = agg_ref[0] + agg_ref[1]
        deg = deg_ref[0, :, 0:1] + deg_ref[1, :, 0:1]
        mean = agg / jnp.maximum(deg, 1.0)
        r = (jnp.dot(mean, wl_ref[...], preferred_element_type=jnp.float32)
             + bl_ref[...]
             + jnp.dot(h_ref[...], wr_ref[...],
                       preferred_element_type=jnp.float32))
        if last:
            m = jnp.max(r, axis=-1, keepdims=True)
            lse = jnp.log(jnp.sum(jnp.exp(r - m), axis=-1, keepdims=True)) + m
            out_ref[...] = r - lse
        else:
            out_ref[...] = jnp.maximum(r, 0.0)

    return pl.pallas_call(
        body,
        out_shape=jax.ShapeDtypeStruct((n_nodes, d), jnp.float32),
        grid=(n_nodes // blk,),
        in_specs=[
            pl.BlockSpec((NC, blk, d), lambda i: (0, i, 0)),
            pl.BlockSpec((NC, blk, d), lambda i: (0, i, 0)),
            pl.BlockSpec((blk, d), lambda i: (i, 0)),
            pl.BlockSpec((d, d), lambda i: (0, 0)),
            pl.BlockSpec((1, d), lambda i: (0, 0)),
            pl.BlockSpec((d, d), lambda i: (0, 0)),
        ],
        out_specs=pl.BlockSpec((blk, d), lambda i: (i, 0)),
    )(aggp, degp, h, wl_t, bl, wr_t)


def kernel(x, edge_index, Wl1, bl1, Wr1, Wl2, bl2, Wr2, Wl3, bl3, Wr3):
    src = edge_index[0].astype(jnp.int32)
    dst = edge_index[1].astype(jnp.int32)
    n_nodes = x.shape[0]
    n_pad = ((n_nodes + 8 * NS - 1) // (8 * NS)) * (8 * NS)
    xp = jnp.pad(x, ((0, n_pad - n_nodes), (0, 0)))

    ones = jnp.ones((n_pad, x.shape[1]), jnp.float32)
    degp = _sc_aggregate(ones, dst, dst)
    xp, degp, ones, src, dst = lax.optimization_barrier(
        (xp, degp, ones, src, dst))
    keep = [degp, xp, ones]
    h = xp
    layers = [(Wl1, bl1, Wr1, False), (Wl2, bl2, Wr2, False),
              (Wl3, bl3, Wr3, True)]
    for wl, bl, wr, last in layers:
        aggp = _sc_aggregate(h, src, dst)
        aggp, h, degp = lax.optimization_barrier((aggp, h, degp))
        h = _tc_combine(aggp, degp, h, wl.T, bl.reshape(1, -1), wr.T, last)
        (h,) = lax.optimization_barrier((h,))
        keep.extend([aggp, h])
    res = lax.optimization_barrier(tuple([h, src, dst] + keep))
    return res[0][:n_nodes]
